# baseline (speedup 1.0000x reference)
import jax
import jax.numpy as jnp
import numpy as np
from jax.experimental import pallas as pl
from jax.experimental.pallas import tpu as pltpu

_C = 1024
_T = 576
_HEADS = 16
_HD = _C // _HEADS
_SZ = 128
_CA = _C + _SZ
_HIGHEST = jax.lax.Precision.HIGHEST


def _split_even_odd(v):
    p, c = v.shape
    r = v.reshape(p // 2, 2, c)
    return r[:, 0, :], r[:, 1, :]


def _tile8(s):
    return jnp.concatenate([s] * (_C // _SZ), axis=1)


def _metric(x):
    m = x[:, 0:_HD]
    for h in range(1, _HEADS):
        m = m + x[:, h * _HD : (h + 1) * _HD]
    m = m / np.float32(_HEADS)
    m2 = m * m
    p8 = m2[:, 0:8]
    for k in range(1, 8):
        p8 = p8 + m2[:, 8 * k : 8 * k + 8]
    q = p8[:, 0:4] + p8[:, 4:8]
    rr = q[:, 0:2] + q[:, 2:4]
    s = rr[:, 0:1] + rr[:, 1:2]
    return m / jnp.sqrt(s)


def _row_of_col(v, n):
    eye = (
        jax.lax.broadcasted_iota(jnp.int32, (n, n), 0)
        == jax.lax.broadcasted_iota(jnp.int32, (n, n), 1)
    ).astype(jnp.float32)
    return jax.lax.dot_general(
        v, eye, (((0,), (0,)), ((), ())), precision=_HIGHEST
    )


def _merge_step(D, acc, half, r, first=False):
    x = D[:, :_C]
    s = D[:, _C:]
    metric = _metric(x)
    a, b = _split_even_odd(metric)
    scores = jax.lax.dot_general(a, b, (((1,), (1,)), ((), ())))
    nm = jnp.max(scores, axis=1, keepdims=True)
    jj = jax.lax.broadcasted_iota(jnp.int32, (half, half), 1)
    ii = jax.lax.broadcasted_iota(jnp.int32, (half, half), 0)
    eq = scores == nm
    jsel = jnp.min(jnp.where(eq, jj, half), axis=1, keepdims=True)
    n2t = jj == jsel
    nm_t = _row_of_col(nm, half)
    before = (nm_t > nm) | ((nm_t == nm) & (jj < ii))
    jself = jsel.astype(jnp.float32)
    samedst = _row_of_col(jself, half) == jself
    occ = jnp.sum(
        (samedst & before).astype(jnp.float32), axis=1, keepdims=True
    )

    if first:
        t_aug = D
    else:
        t_aug = jnp.concatenate([x * _tile8(s), s], axis=1)
    te, to = _split_even_odd(t_aug)
    if r < half:
        rank = jnp.sum(before.astype(jnp.float32), axis=1, keepdims=True)
        is_src = rank < r
        slotmat = jnp.where(n2t & is_src, occ, -1.0)
        n_slots = jnp.max(jnp.where(is_src, occ, -1.0)).astype(jnp.int32) + 1
    else:
        slotmat = jnp.where(n2t, occ, -1.0)
        n_slots = jnp.max(occ).astype(jnp.int32) + 1

    acc[0:half, :] = to

    te_hi = te.astype(jnp.bfloat16)
    r1 = te - te_hi.astype(jnp.float32)
    te_mid = r1.astype(jnp.bfloat16)
    te_lo = (r1 - te_mid.astype(jnp.float32)).astype(jnp.bfloat16)

    def body(t, carry):
        pt = (slotmat == t.astype(jnp.float32)).astype(jnp.bfloat16)
        dims = (((0,), (0,)), ((), ()))
        c_hi = jax.lax.dot_general(
            pt, te_hi, dims, preferred_element_type=jnp.float32
        )
        c_mid = jax.lax.dot_general(
            pt, te_mid, dims, preferred_element_type=jnp.float32
        )
        c_lo = jax.lax.dot_general(
            pt, te_lo, dims, preferred_element_type=jnp.float32
        )
        acc[0:half, :] = acc[0:half, :] + ((c_hi + c_mid) + c_lo)
        return carry

    jax.lax.fori_loop(0, n_slots, body, 0)
    merged = acc[0:half, :]
    if r < half:
        rank_t = _row_of_col(rank, half)
        kk = jax.lax.broadcasted_iota(jnp.int32, (half - r, half), 0) + r
        u = (rank_t == kk.astype(jnp.float32)).astype(jnp.float32)
        unm = jax.lax.dot(u, te, precision=_HIGHEST)
        merged = jnp.concatenate([unm, merged], axis=0)
    ss = merged[:, _C:]
    newx = merged[:, :_C] / _tile8(ss)
    return jnp.concatenate([newx, ss], axis=1)


def _fused_kernel(x_ref, w1_ref, b1_ref, w2_ref, b2_ref, o_ref, acc):
    x = x_ref[0]
    D = jnp.concatenate([x, jnp.ones((_T, _SZ), jnp.float32)], axis=1)
    D = _merge_step(D, acc, 288, 288, first=True)
    D = _merge_step(D, acc, 144, 144)
    D = _merge_step(D, acc, 72, 72)
    D = _merge_step(D, acc, 36, 8)
    y = D[:, :_C]
    h = jax.lax.dot(y, w1_ref[...]) + b1_ref[...]
    h = 0.5 * h * (1.0 + jax.lax.erf(h * np.float32(1.0 / np.sqrt(2.0))))
    out = jax.lax.dot(h, w2_ref[...]) + b2_ref[...]
    o_ref[0] = out


@jax.jit
def kernel(x, W1, b1, W2, b2):
    B, T, C = x.shape
    return pl.pallas_call(
        _fused_kernel,
        grid=(B,),
        in_specs=[
            pl.BlockSpec((1, T, C), lambda i: (i, 0, 0)),
            pl.BlockSpec((C, C), lambda i: (0, 0)),
            pl.BlockSpec((1, C), lambda i: (0, 0)),
            pl.BlockSpec((C, C), lambda i: (0, 0)),
            pl.BlockSpec((1, C), lambda i: (0, 0)),
        ],
        out_specs=pl.BlockSpec((1, 64, C), lambda i: (i, 0, 0)),
        out_shape=jax.ShapeDtypeStruct((B, 64, C), x.dtype),
        scratch_shapes=[pltpu.VMEM((288, _CA), jnp.float32)],
        compiler_params=pltpu.CompilerParams(
            dimension_semantics=("parallel",)
        ),
    )(x, W1, b1.reshape(1, C), W2, b2.reshape(1, C))


# speedup vs baseline: 1.0026x; 1.0026x over previous
import jax
import jax.numpy as jnp
import numpy as np
from jax.experimental import pallas as pl
from jax.experimental.pallas import tpu as pltpu

_C = 1024
_T = 576
_HEADS = 16
_HD = _C // _HEADS
_SZ = 128
_CA = _C + _SZ
_HIGHEST = jax.lax.Precision.HIGHEST


def _split_even_odd(v):
    p, c = v.shape
    r = v.reshape(p // 2, 2, c)
    return r[:, 0, :], r[:, 1, :]


def _tile8(s):
    return jnp.concatenate([s] * (_C // _SZ), axis=1)


def _metric(x):
    m = x[:, 0:_HD]
    for h in range(1, _HEADS):
        m = m + x[:, h * _HD : (h + 1) * _HD]
    m = m / np.float32(_HEADS)
    m2 = m * m
    p8 = m2[:, 0:8]
    for k in range(1, 8):
        p8 = p8 + m2[:, 8 * k : 8 * k + 8]
    q = p8[:, 0:4] + p8[:, 4:8]
    rr = q[:, 0:2] + q[:, 2:4]
    s = rr[:, 0:1] + rr[:, 1:2]
    return m / jnp.sqrt(s)


def _row_of_col(v, n):
    eye = (
        jax.lax.broadcasted_iota(jnp.int32, (n, n), 0)
        == jax.lax.broadcasted_iota(jnp.int32, (n, n), 1)
    ).astype(jnp.float32)
    return jax.lax.dot_general(
        v, eye, (((0,), (0,)), ((), ())), precision=_HIGHEST
    )


def _merge_step(D, acc, half, r, first=False):
    x = D[:, :_C]
    s = D[:, _C:]
    metric = _metric(x)
    a, b = _split_even_odd(metric)
    scores = jax.lax.dot_general(a, b, (((1,), (1,)), ((), ())))
    nm = jnp.max(scores, axis=1, keepdims=True)
    jj = jax.lax.broadcasted_iota(jnp.int32, (half, half), 1)
    ii = jax.lax.broadcasted_iota(jnp.int32, (half, half), 0)
    eq = scores == nm
    jsel = jnp.min(jnp.where(eq, jj, half), axis=1, keepdims=True)
    n2t = jj == jsel
    nm_t = _row_of_col(nm, half)
    before = (nm_t > nm) | ((nm_t == nm) & (jj < ii))
    jself = jsel.astype(jnp.float32)
    samedst = _row_of_col(jself, half) == jself
    occ = jnp.sum(
        (samedst & before).astype(jnp.float32), axis=1, keepdims=True
    )

    if first:
        t_aug = D
    else:
        t_aug = jnp.concatenate([x * _tile8(s), s], axis=1)
    te, to = _split_even_odd(t_aug)
    if r < half:
        rank = jnp.sum(before.astype(jnp.float32), axis=1, keepdims=True)
        is_src = rank < r
        slotmat = jnp.where(n2t & is_src, occ, -1.0)
        n_slots = jnp.max(jnp.where(is_src, occ, -1.0)).astype(jnp.int32) + 1
    else:
        slotmat = jnp.where(n2t, occ, -1.0)
        n_slots = jnp.max(occ).astype(jnp.int32) + 1

    acc[0:half, :] = to

    te_hi = te.astype(jnp.bfloat16)
    r1 = te - te_hi.astype(jnp.float32)
    te_mid = r1.astype(jnp.bfloat16)
    te_lo = (r1 - te_mid.astype(jnp.float32)).astype(jnp.bfloat16)

    def body(t, carry):
        pt = (slotmat == t.astype(jnp.float32)).astype(jnp.bfloat16)
        dims = (((0,), (0,)), ((), ()))
        c_hi = jax.lax.dot_general(
            pt, te_hi, dims, preferred_element_type=jnp.float32
        )
        c_mid = jax.lax.dot_general(
            pt, te_mid, dims, preferred_element_type=jnp.float32
        )
        c_lo = jax.lax.dot_general(
            pt, te_lo, dims, preferred_element_type=jnp.float32
        )
        acc[0:half, :] = acc[0:half, :] + ((c_hi + c_mid) + c_lo)
        return carry

    jax.lax.fori_loop(0, n_slots, body, 0)
    merged = acc[0:half, :]
    if r < half:
        rank_t = _row_of_col(rank, half)
        kk = jax.lax.broadcasted_iota(jnp.int32, (half - r, half), 0) + r
        u = (rank_t == kk.astype(jnp.float32)).astype(jnp.float32)
        unm = jax.lax.dot(u, te, precision=_HIGHEST)
        merged = jnp.concatenate([unm, merged], axis=0)
    ss = merged[:, _C:]
    newx = merged[:, :_C] / _tile8(ss)
    return jnp.concatenate([newx, ss], axis=1)


def _fused_kernel(x_ref, w1_ref, b1_ref, w2_ref, b2_ref, o_ref, acc):
    x = x_ref[0]
    D = jnp.concatenate([x, jnp.ones((_T, _SZ), jnp.float32)], axis=1)
    D = _merge_step(D, acc, 288, 288, first=True)
    D = _merge_step(D, acc, 144, 144)
    D = _merge_step(D, acc, 72, 72)
    D = _merge_step(D, acc, 36, 8)
    y = D[:, :_C]
    h = jax.lax.dot(y, w1_ref[...]) + b1_ref[...]
    h = 0.5 * h * (1.0 + jax.lax.erf(h * np.float32(1.0 / np.sqrt(2.0))))
    out = jax.lax.dot(h, w2_ref[...]) + b2_ref[...]
    o_ref[0] = out


@jax.jit
def kernel(x, W1, b1, W2, b2):
    B, T, C = x.shape
    return pl.pallas_call(
        _fused_kernel,
        grid=(B,),
        in_specs=[
            pl.BlockSpec((1, T, C), lambda i: (i, 0, 0)),
            pl.BlockSpec((C, C), lambda i: (0, 0)),
            pl.BlockSpec((1, C), lambda i: (0, 0)),
            pl.BlockSpec((C, C), lambda i: (0, 0)),
            pl.BlockSpec((1, C), lambda i: (0, 0)),
        ],
        out_specs=pl.BlockSpec((1, 64, C), lambda i: (i, 0, 0)),
        out_shape=jax.ShapeDtypeStruct((B, 64, C), x.dtype),
        scratch_shapes=[pltpu.VMEM((288, _CA), jnp.float32)],
        compiler_params=pltpu.CompilerParams(
            dimension_semantics=("arbitrary",)
        ),
    )(x, W1, b1.reshape(1, C), W2, b2.reshape(1, C))


# speedup vs baseline: 1.5561x; 1.5521x over previous
import jax
import jax.numpy as jnp
import numpy as np
from jax.experimental import pallas as pl
from jax.experimental.pallas import tpu as pltpu

_C = 1024
_T = 576
_HEADS = 16
_HD = _C // _HEADS
_SZ = 128
_CA = _C + _SZ
_HIGHEST = jax.lax.Precision.HIGHEST


def _split_even_odd(v):
    p, c = v.shape
    r = v.reshape(p // 2, 2, c)
    return r[:, 0, :], r[:, 1, :]


def _tile8(s):
    return jnp.concatenate([s] * (_C // _SZ), axis=1)


def _metric(x):
    m = x[:, 0:_HD]
    for h in range(1, _HEADS):
        m = m + x[:, h * _HD : (h + 1) * _HD]
    m = m / np.float32(_HEADS)
    m2 = m * m
    p8 = m2[:, 0:8]
    for k in range(1, 8):
        p8 = p8 + m2[:, 8 * k : 8 * k + 8]
    q = p8[:, 0:4] + p8[:, 4:8]
    rr = q[:, 0:2] + q[:, 2:4]
    s = rr[:, 0:1] + rr[:, 1:2]
    return m / jnp.sqrt(s)


def _row_of_col(v, n):
    eye = (
        jax.lax.broadcasted_iota(jnp.int32, (n, n), 0)
        == jax.lax.broadcasted_iota(jnp.int32, (n, n), 1)
    ).astype(jnp.float32)
    return jax.lax.dot_general(
        v, eye, (((0,), (0,)), ((), ())), precision=_HIGHEST
    )


def _merge_step(D, acc, half, r, first=False):
    x = D[:, :_C]
    s = D[:, _C:]
    metric = _metric(x)
    a, b = _split_even_odd(metric)
    scores = jax.lax.dot_general(a, b, (((1,), (1,)), ((), ())))
    nm = jnp.max(scores, axis=1, keepdims=True)
    jj = jax.lax.broadcasted_iota(jnp.int32, (half, half), 1)
    ii = jax.lax.broadcasted_iota(jnp.int32, (half, half), 0)
    eq = scores == nm
    jsel = jnp.min(jnp.where(eq, jj, half), axis=1, keepdims=True)
    n2t = jj == jsel
    nm_t = _row_of_col(nm, half)
    before = (nm_t > nm) | ((nm_t == nm) & (jj < ii))
    jself = jsel.astype(jnp.float32)
    samedst = _row_of_col(jself, half) == jself
    occ = jnp.sum(
        (samedst & before).astype(jnp.float32), axis=1, keepdims=True
    )

    if first:
        t_aug = D
    else:
        t_aug = jnp.concatenate([x * _tile8(s), s], axis=1)
    te, to = _split_even_odd(t_aug)
    jsel_row = _row_of_col(jself, half)
    n2tt = ii == jsel_row
    occ_row = _row_of_col(occ, half)
    if r < half:
        rank = jnp.sum(before.astype(jnp.float32), axis=1, keepdims=True)
        rank_row = _row_of_col(rank, half)
        slotmat = jnp.where(n2tt & (rank_row < r), occ_row, -1.0)
        n_slots = jnp.max(jnp.where(rank < r, occ, -1.0)).astype(jnp.int32) + 1
    else:
        slotmat = jnp.where(n2tt, occ_row, -1.0)
        n_slots = jnp.max(occ).astype(jnp.int32) + 1

    acc[0:half, :] = to

    te_hi = te.astype(jnp.bfloat16)
    r1 = te - te_hi.astype(jnp.float32)
    te_mid = r1.astype(jnp.bfloat16)
    te_lo = (r1 - te_mid.astype(jnp.float32)).astype(jnp.bfloat16)
    te3 = jnp.concatenate([te_hi, te_mid, te_lo], axis=0)
    slotmat3 = jnp.concatenate([slotmat] * 3, axis=1)

    def body(t, carry):
        pt3 = (slotmat3 == t.astype(jnp.float32)).astype(jnp.bfloat16)
        contrib = jax.lax.dot_general(
            pt3, te3, (((1,), (0,)), ((), ())),
            preferred_element_type=jnp.float32,
        )
        acc[0:half, :] = acc[0:half, :] + contrib
        return carry

    jax.lax.fori_loop(0, n_slots, body, 0)
    merged = acc[0:half, :]
    if r < half:
        rank_t = _row_of_col(rank, half)
        kk = jax.lax.broadcasted_iota(jnp.int32, (half - r, half), 0) + r
        u = (rank_t == kk.astype(jnp.float32)).astype(jnp.float32)
        unm = jax.lax.dot(u, te, precision=_HIGHEST)
        merged = jnp.concatenate([unm, merged], axis=0)
    ss = merged[:, _C:]
    newx = merged[:, :_C] / _tile8(ss)
    return jnp.concatenate([newx, ss], axis=1)


def _fused_kernel(x_ref, w1_ref, b1_ref, w2_ref, b2_ref, o_ref, acc):
    x = x_ref[0]
    D = jnp.concatenate([x, jnp.ones((_T, _SZ), jnp.float32)], axis=1)
    D = _merge_step(D, acc, 288, 288, first=True)
    D = _merge_step(D, acc, 144, 144)
    D = _merge_step(D, acc, 72, 72)
    D = _merge_step(D, acc, 36, 8)
    y = D[:, :_C]
    h = jax.lax.dot(y, w1_ref[...]) + b1_ref[...]
    h = 0.5 * h * (1.0 + jax.lax.erf(h * np.float32(1.0 / np.sqrt(2.0))))
    out = jax.lax.dot(h, w2_ref[...]) + b2_ref[...]
    o_ref[0] = out


@jax.jit
def kernel(x, W1, b1, W2, b2):
    B, T, C = x.shape
    return pl.pallas_call(
        _fused_kernel,
        grid=(B,),
        in_specs=[
            pl.BlockSpec((1, T, C), lambda i: (i, 0, 0)),
            pl.BlockSpec((C, C), lambda i: (0, 0)),
            pl.BlockSpec((1, C), lambda i: (0, 0)),
            pl.BlockSpec((C, C), lambda i: (0, 0)),
            pl.BlockSpec((1, C), lambda i: (0, 0)),
        ],
        out_specs=pl.BlockSpec((1, 64, C), lambda i: (i, 0, 0)),
        out_shape=jax.ShapeDtypeStruct((B, 64, C), x.dtype),
        scratch_shapes=[pltpu.VMEM((288, _CA), jnp.float32)],
        compiler_params=pltpu.CompilerParams(
            dimension_semantics=("arbitrary",)
        ),
    )(x, W1, b1.reshape(1, C), W2, b2.reshape(1, C))


# speedup vs baseline: 1.6502x; 1.0605x over previous
import jax
import jax.numpy as jnp
import numpy as np
from jax.experimental import pallas as pl
from jax.experimental.pallas import tpu as pltpu

_C = 1024
_T = 576
_HEADS = 16
_HD = _C // _HEADS
_SZ = 128
_CA = _C + _SZ
_HIGHEST = jax.lax.Precision.HIGHEST


def _split_even_odd(v):
    p, c = v.shape
    r = v.reshape(p // 2, 2, c)
    return r[:, 0, :], r[:, 1, :]


def _tile8(s):
    return jnp.concatenate([s] * (_C // _SZ), axis=1)


def _metric(x):
    m = x[:, 0:_HD]
    for h in range(1, _HEADS):
        m = m + x[:, h * _HD : (h + 1) * _HD]
    m = m / np.float32(_HEADS)
    m2 = m * m
    p8 = m2[:, 0:8]
    for k in range(1, 8):
        p8 = p8 + m2[:, 8 * k : 8 * k + 8]
    q = p8[:, 0:4] + p8[:, 4:8]
    rr = q[:, 0:2] + q[:, 2:4]
    s = rr[:, 0:1] + rr[:, 1:2]
    return m / jnp.sqrt(s)


def _row_of_col(v, n):
    eye = (
        jax.lax.broadcasted_iota(jnp.int32, (n, n), 0)
        == jax.lax.broadcasted_iota(jnp.int32, (n, n), 1)
    ).astype(jnp.float32)
    return jax.lax.dot_general(
        v, eye, (((0,), (0,)), ((), ())), precision=_HIGHEST
    )


def _merge_step(D, acc, half, r, first=False):
    x = D[:, :_C]
    s = D[:, _C:]
    metric = _metric(x)
    a, b = _split_even_odd(metric)
    scores = jax.lax.dot_general(a, b, (((1,), (1,)), ((), ())))
    nm = jnp.max(scores, axis=1, keepdims=True)
    jj = jax.lax.broadcasted_iota(jnp.int32, (half, half), 1)
    ii = jax.lax.broadcasted_iota(jnp.int32, (half, half), 0)
    eq = scores == nm
    jsel = jnp.min(jnp.where(eq, jj, half), axis=1, keepdims=True)
    n2t = jj == jsel
    nm_t = _row_of_col(nm, half)
    before = (nm_t > nm) | ((nm_t == nm) & (jj < ii))
    jself = jsel.astype(jnp.float32)
    samedst = _row_of_col(jself, half) == jself
    occ = jnp.sum(
        (samedst & before).astype(jnp.float32), axis=1, keepdims=True
    )

    if first:
        t_aug = D
    else:
        t_aug = jnp.concatenate([x * _tile8(s), s], axis=1)
    te, to = _split_even_odd(t_aug)
    jsel_row = _row_of_col(jself, half)
    n2tt = ii == jsel_row
    occ_row = _row_of_col(occ, half)
    if r < half:
        rank = jnp.sum(before.astype(jnp.float32), axis=1, keepdims=True)
        rank_row = _row_of_col(rank, half)
        slotmat = jnp.where(n2tt & (rank_row < r), occ_row, -1.0)
        n_slots = jnp.max(jnp.where(rank < r, occ, -1.0)).astype(jnp.int32) + 1
    else:
        slotmat = jnp.where(n2tt, occ_row, -1.0)
        n_slots = jnp.max(occ).astype(jnp.int32) + 1

    sel = jnp.where(slotmat >= 0.0, 1.0, 0.0)
    ss_add = jax.lax.dot_general(
        sel, te[:, _C:], (((1,), (0,)), ((), ())), precision=_HIGHEST
    )
    acc[0:half, _C:] = to[:, _C:] + ss_add
    acc[0:half, 0:_C] = to[:, 0:_C]

    ted = te[:, 0:_C]
    te_hi = ted.astype(jnp.bfloat16)
    r1 = ted - te_hi.astype(jnp.float32)
    te_mid = r1.astype(jnp.bfloat16)
    te_lo = (r1 - te_mid.astype(jnp.float32)).astype(jnp.bfloat16)
    te3 = jnp.concatenate([te_hi, te_mid, te_lo], axis=0)
    slotmat3 = jnp.concatenate([slotmat] * 3, axis=1)

    def one_slot(t):
        pt3 = (slotmat3 == t).astype(jnp.bfloat16)
        return jax.lax.dot_general(
            pt3, te3, (((1,), (0,)), ((), ())),
            preferred_element_type=jnp.float32,
        )

    def body(u, carry):
        t = (2 * u).astype(jnp.float32)
        c0 = one_slot(t)
        c1 = one_slot(t + 1.0)
        acc[0:half, 0:_C] = (acc[0:half, 0:_C] + c0) + c1
        return carry

    jax.lax.fori_loop(0, (n_slots + 1) // 2, body, 0)
    merged = acc[0:half, :]
    if r < half:
        rank_t = _row_of_col(rank, half)
        kk = jax.lax.broadcasted_iota(jnp.int32, (half - r, half), 0) + r
        u = (rank_t == kk.astype(jnp.float32)).astype(jnp.float32)
        unm = jax.lax.dot(u, te, precision=_HIGHEST)
        merged = jnp.concatenate([unm, merged], axis=0)
    ss = merged[:, _C:]
    newx = merged[:, :_C] / _tile8(ss)
    return jnp.concatenate([newx, ss], axis=1)


def _fused_kernel(x_ref, w1_ref, b1_ref, w2_ref, b2_ref, o_ref, acc):
    x = x_ref[0]
    D = jnp.concatenate([x, jnp.ones((_T, _SZ), jnp.float32)], axis=1)
    D = _merge_step(D, acc, 288, 288, first=True)
    D = _merge_step(D, acc, 144, 144)
    D = _merge_step(D, acc, 72, 72)
    D = _merge_step(D, acc, 36, 8)
    y = D[:, :_C]
    h = jax.lax.dot(y, w1_ref[...]) + b1_ref[...]
    h = 0.5 * h * (1.0 + jax.lax.erf(h * np.float32(1.0 / np.sqrt(2.0))))
    out = jax.lax.dot(h, w2_ref[...]) + b2_ref[...]
    o_ref[0] = out


@jax.jit
def kernel(x, W1, b1, W2, b2):
    B, T, C = x.shape
    return pl.pallas_call(
        _fused_kernel,
        grid=(B,),
        in_specs=[
            pl.BlockSpec((1, T, C), lambda i: (i, 0, 0)),
            pl.BlockSpec((C, C), lambda i: (0, 0)),
            pl.BlockSpec((1, C), lambda i: (0, 0)),
            pl.BlockSpec((C, C), lambda i: (0, 0)),
            pl.BlockSpec((1, C), lambda i: (0, 0)),
        ],
        out_specs=pl.BlockSpec((1, 64, C), lambda i: (i, 0, 0)),
        out_shape=jax.ShapeDtypeStruct((B, 64, C), x.dtype),
        scratch_shapes=[pltpu.VMEM((288, _CA), jnp.float32)],
        compiler_params=pltpu.CompilerParams(
            dimension_semantics=("arbitrary",)
        ),
    )(x, W1, b1.reshape(1, C), W2, b2.reshape(1, C))


# speedup vs baseline: 1.6550x; 1.0029x over previous
import jax
import jax.numpy as jnp
import numpy as np
from jax.experimental import pallas as pl
from jax.experimental.pallas import tpu as pltpu

_C = 1024
_T = 576
_HEADS = 16
_HD = _C // _HEADS
_SZ = 128
_CA = _C + _SZ
_HIGHEST = jax.lax.Precision.HIGHEST


def _split_even_odd(v):
    p, c = v.shape
    r = v.reshape(p // 2, 2, c)
    return r[:, 0, :], r[:, 1, :]


def _tile8(s):
    return jnp.concatenate([s] * (_C // _SZ), axis=1)


def _metric(x):
    m = x[:, 0:_HD]
    for h in range(1, _HEADS):
        m = m + x[:, h * _HD : (h + 1) * _HD]
    m = m / np.float32(_HEADS)
    m2 = m * m
    p8 = m2[:, 0:8]
    for k in range(1, 8):
        p8 = p8 + m2[:, 8 * k : 8 * k + 8]
    q = p8[:, 0:4] + p8[:, 4:8]
    rr = q[:, 0:2] + q[:, 2:4]
    s = rr[:, 0:1] + rr[:, 1:2]
    return m / jnp.sqrt(s)


def _row_of_col(v, n):
    eye = (
        jax.lax.broadcasted_iota(jnp.int32, (n, n), 0)
        == jax.lax.broadcasted_iota(jnp.int32, (n, n), 1)
    ).astype(jnp.float32)
    return jax.lax.dot_general(
        v, eye, (((0,), (0,)), ((), ())), precision=_HIGHEST
    )


def _merge_step(D, acc, te3, half, r, first=False):
    x = D[:, :_C]
    s = D[:, _C:]
    metric = _metric(x)
    a, b = _split_even_odd(metric)
    scores = jax.lax.dot_general(a, b, (((1,), (1,)), ((), ())))
    nm = jnp.max(scores, axis=1, keepdims=True)
    jj = jax.lax.broadcasted_iota(jnp.int32, (half, half), 1)
    ii = jax.lax.broadcasted_iota(jnp.int32, (half, half), 0)
    eq = scores == nm
    jsel = jnp.min(jnp.where(eq, jj, half), axis=1, keepdims=True)
    n2t = jj == jsel
    nm_t = _row_of_col(nm, half)
    before = (nm_t > nm) | ((nm_t == nm) & (jj < ii))
    jself = jsel.astype(jnp.float32)
    samedst = _row_of_col(jself, half) == jself
    occ = jnp.sum(
        (samedst & before).astype(jnp.float32), axis=1, keepdims=True
    )

    if first:
        t_aug = D
    else:
        t_aug = jnp.concatenate([x * _tile8(s), s], axis=1)
    te, to = _split_even_odd(t_aug)
    jsel_row = _row_of_col(jself, half)
    n2tt = ii == jsel_row
    occ_row = _row_of_col(occ, half)
    if r < half:
        rank = jnp.sum(before.astype(jnp.float32), axis=1, keepdims=True)
        rank_row = _row_of_col(rank, half)
        slotmat = jnp.where(n2tt & (rank_row < r), occ_row, -1.0)
        n_slots = jnp.max(jnp.where(rank < r, occ, -1.0)).astype(jnp.int32) + 1
    else:
        slotmat = jnp.where(n2tt, occ_row, -1.0)
        n_slots = jnp.max(occ).astype(jnp.int32) + 1

    sel = jnp.where(slotmat >= 0.0, 1.0, 0.0)
    ss_add = jax.lax.dot_general(
        sel, te[:, _C:], (((1,), (0,)), ((), ())), precision=_HIGHEST
    )
    acc[0:half, _C:] = to[:, _C:] + ss_add
    acc[0:half, 0:_C] = to[:, 0:_C]

    ted = te[:, 0:_C]
    te_hi = ted.astype(jnp.bfloat16)
    r1 = ted - te_hi.astype(jnp.float32)
    te_mid = r1.astype(jnp.bfloat16)
    te_lo = (r1 - te_mid.astype(jnp.float32)).astype(jnp.bfloat16)
    te3[0:half, :] = te_hi
    te3[half : 2 * half, :] = te_mid
    te3[2 * half : 3 * half, :] = te_lo
    slotmat3 = jnp.concatenate([slotmat] * 3, axis=1)

    def one_slot(t):
        pt3 = (slotmat3 == t).astype(jnp.bfloat16)
        return jax.lax.dot_general(
            pt3, te3[0 : 3 * half, :], (((1,), (0,)), ((), ())),
            preferred_element_type=jnp.float32,
        )

    def body(u, carry):
        t = (2 * u).astype(jnp.float32)
        c0 = one_slot(t)
        c1 = one_slot(t + 1.0)
        acc[0:half, 0:_C] = (acc[0:half, 0:_C] + c0) + c1
        return carry

    jax.lax.fori_loop(0, (n_slots + 1) // 2, body, 0)
    merged = acc[0:half, :]
    if r < half:
        rank_t = _row_of_col(rank, half)
        kk = jax.lax.broadcasted_iota(jnp.int32, (half - r, half), 0) + r
        u = (rank_t == kk.astype(jnp.float32)).astype(jnp.float32)
        unm = jax.lax.dot(u, te, precision=_HIGHEST)
        merged = jnp.concatenate([unm, merged], axis=0)
    ss = merged[:, _C:]
    newx = merged[:, :_C] / _tile8(ss)
    return jnp.concatenate([newx, ss], axis=1)


def _fused_kernel(x_ref, w1_ref, b1_ref, w2_ref, b2_ref, o_ref, acc, te3):
    x = x_ref[0]
    D = jnp.concatenate([x, jnp.ones((_T, _SZ), jnp.float32)], axis=1)
    D = _merge_step(D, acc, te3, 288, 288, first=True)
    D = _merge_step(D, acc, te3, 144, 144)
    D = _merge_step(D, acc, te3, 72, 72)
    D = _merge_step(D, acc, te3, 36, 8)
    y = D[:, :_C]
    h = jax.lax.dot(y, w1_ref[...]) + b1_ref[...]
    h = 0.5 * h * (1.0 + jax.lax.erf(h * np.float32(1.0 / np.sqrt(2.0))))
    out = jax.lax.dot(h, w2_ref[...]) + b2_ref[...]
    o_ref[0] = out


@jax.jit
def kernel(x, W1, b1, W2, b2):
    B, T, C = x.shape
    return pl.pallas_call(
        _fused_kernel,
        grid=(B,),
        in_specs=[
            pl.BlockSpec((1, T, C), lambda i: (i, 0, 0)),
            pl.BlockSpec((C, C), lambda i: (0, 0)),
            pl.BlockSpec((1, C), lambda i: (0, 0)),
            pl.BlockSpec((C, C), lambda i: (0, 0)),
            pl.BlockSpec((1, C), lambda i: (0, 0)),
        ],
        out_specs=pl.BlockSpec((1, 64, C), lambda i: (i, 0, 0)),
        out_shape=jax.ShapeDtypeStruct((B, 64, C), x.dtype),
        scratch_shapes=[
            pltpu.VMEM((288, _CA), jnp.float32),
            pltpu.VMEM((3 * 288, _C), jnp.bfloat16),
        ],
        compiler_params=pltpu.CompilerParams(
            dimension_semantics=("arbitrary",)
        ),
    )(x, W1, b1.reshape(1, C), W2, b2.reshape(1, C))
